# baseline (device time: 9871 ns/iter reference)
import jax
import jax.numpy as jnp
from jax import lax
from jax.experimental import pallas as pl
from jax.experimental.pallas import tpu as pltpu

N_DEV = 4
TAPS = 4
HALO = TAPS - 1


def kernel(x, k):
    b, s, c = x.shape

    def body(x_ref, k_ref, out_ref, send_buf, halo_ref, pad_ref, send_sem, recv_sem):
        my = lax.axis_index("i")
        left = (my - 1) % N_DEV
        right = (my + 1) % N_DEV

        barrier_sem = pltpu.get_barrier_semaphore()
        for nbr in (left, right):
            pl.semaphore_signal(
                barrier_sem, inc=1,
                device_id=(nbr,), device_id_type=pl.DeviceIdType.MESH,
            )
        pl.semaphore_wait(barrier_sem, 2)

        send_buf[...] = x_ref[:, s - HALO:, :]
        rdma = pltpu.make_async_remote_copy(
            src_ref=send_buf,
            dst_ref=halo_ref,
            send_sem=send_sem,
            recv_sem=recv_sem,
            device_id=(right,),
            device_id_type=pl.DeviceIdType.MESH,
        )
        rdma.start()

        pad_ref[:, HALO:, :] = x_ref[...]

        rdma.wait()
        hv = halo_ref[...]
        pad_ref[:, :HALO, :] = jnp.where(my == 0, jnp.zeros_like(hv), hv)

        kv = k_ref[...]
        acc = pad_ref[:, 0:s, :] * kv[0]
        for t in range(1, TAPS):
            acc += pad_ref[:, t:t + s, :] * kv[t]
        out_ref[...] = acc / (1.0 + jnp.exp(-acc))

    return pl.pallas_call(
        body,
        out_shape=jax.ShapeDtypeStruct((b, s, c), x.dtype),
        in_specs=[
            pl.BlockSpec(memory_space=pltpu.VMEM),
            pl.BlockSpec(memory_space=pltpu.VMEM),
        ],
        out_specs=pl.BlockSpec(memory_space=pltpu.VMEM),
        scratch_shapes=[
            pltpu.VMEM((b, HALO, c), x.dtype),
            pltpu.VMEM((b, HALO, c), x.dtype),
            pltpu.VMEM((b, s + HALO, c), x.dtype),
            pltpu.SemaphoreType.DMA,
            pltpu.SemaphoreType.DMA,
        ],
        compiler_params=pltpu.CompilerParams(collective_id=0),
    )(x, k)


# device time: 7365 ns/iter; 1.3403x vs baseline; 1.3403x over previous
import jax
import jax.numpy as jnp
from jax import lax
from jax.experimental import pallas as pl
from jax.experimental.pallas import tpu as pltpu

N_DEV = 4
TAPS = 4
HALO = TAPS - 1


def kernel(x, k):
    b, s, c = x.shape

    def body(x_ref, k_ref, out_ref, send_buf, halo_ref, pad_ref, send_sem, recv_sem):
        my = lax.axis_index("i")
        left = (my - 1) % N_DEV
        right = (my + 1) % N_DEV

        barrier_sem = pltpu.get_barrier_semaphore()
        for nbr in (left, right):
            pl.semaphore_signal(
                barrier_sem, inc=1,
                device_id=(nbr,), device_id_type=pl.DeviceIdType.MESH,
            )
        pl.semaphore_wait(barrier_sem, 2)

        send_buf[...] = x_ref[:, s - HALO:, :]
        rdma = pltpu.make_async_remote_copy(
            src_ref=send_buf,
            dst_ref=halo_ref,
            send_sem=send_sem,
            recv_sem=recv_sem,
            device_id=(right,),
            device_id_type=pl.DeviceIdType.MESH,
        )
        rdma.start()

        pad_ref[:, :HALO, :] = jnp.zeros((b, HALO, c), x_ref.dtype)
        pad_ref[:, HALO:, :] = x_ref[...]

        kv = k_ref[...]
        acc = pad_ref[:, 0:s, :] * kv[0]
        for t in range(1, TAPS):
            acc += pad_ref[:, t:t + s, :] * kv[t]
        out_ref[...] = acc / (1.0 + jnp.exp(-acc))

        rdma.wait()
        hv = halo_ref[...]
        hv = jnp.where(my == 0, jnp.zeros_like(hv), hv)
        rows = []
        for j in range(HALO):
            r = hv[:, j:j + 1, :] * kv[0]
            for t in range(1, HALO - j):
                r += hv[:, j + t:j + t + 1, :] * kv[t]
            rows.append(r)
        corr = jnp.concatenate(rows, axis=1)
        head = acc[:, :HALO, :] + corr
        out_ref[:, :HALO, :] = head / (1.0 + jnp.exp(-head))

    return pl.pallas_call(
        body,
        out_shape=jax.ShapeDtypeStruct((b, s, c), x.dtype),
        in_specs=[
            pl.BlockSpec(memory_space=pltpu.VMEM),
            pl.BlockSpec(memory_space=pltpu.VMEM),
        ],
        out_specs=pl.BlockSpec(memory_space=pltpu.VMEM),
        scratch_shapes=[
            pltpu.VMEM((b, HALO, c), x.dtype),
            pltpu.VMEM((b, HALO, c), x.dtype),
            pltpu.VMEM((b, s + HALO, c), x.dtype),
            pltpu.SemaphoreType.DMA,
            pltpu.SemaphoreType.DMA,
        ],
        compiler_params=pltpu.CompilerParams(collective_id=0),
    )(x, k)
